# baseline (speedup 1.0000x reference)
import dataclasses
import functools

import jax
import jax.numpy as jnp
from jax import lax
from jax.experimental import pallas as pl
from jax.experimental.pallas import tpu as pltpu
from jax.experimental.pallas import tpu_sc as plsc

B = 8
N = 1024
E = 4096
EQ = E // 4
NQ = N // 4
GAMMA = 1.0
LAM = 1.0
N_ITER = 10
STEP = 0.1
EPS = 1e-6
L = 16
NB = 128

_N_BISECT = 0
_N_NEWTON = 2


def _rsqrt(a):
    i = plsc.bitcast(a, jnp.int32)
    i = jnp.int32(0x5F3759DF) - lax.shift_right_arithmetic(i, 1)
    r = plsc.bitcast(i, jnp.float32)
    h = 0.5 * a
    for _ in range(3):
        r = r * (1.5 - h * r * r)
    return r


def _recip(a):
    av = jnp.full((L,), 1.0, jnp.float32) * a
    i = jnp.int32(0x7EF311C3) - plsc.bitcast(av, jnp.int32)
    r = plsc.bitcast(i, jnp.float32)
    for _ in range(4):
        r = r * (2.0 - av * r)
    return jnp.max(r)


def _sc_body(x_hbm, se_hbm, out_hbm, xv, y, agg, t1, parts, src, dst, shared, sem):
    cid = lax.axis_index("c")
    sid = lax.axis_index("s")
    lg = lax.shift_right_logical(sid, 2)
    q = lax.bitwise_and(sid, 3)
    g = cid * 4 + lg

    hs = [
        pltpu.async_copy(x_hbm.at[pl.ds(g * N, N)], xv, sem),
        pltpu.async_copy(x_hbm.at[pl.ds(g * N, N)], y, sem),
        pltpu.async_copy(se_hbm.at[g, 0, pl.ds(q * EQ, EQ)], src, sem),
        pltpu.async_copy(se_hbm.at[g, 1, pl.ds(q * EQ, EQ)], dst, sem),
    ]
    for h in hs:
        h.wait()

    @plsc.parallel_loop(0, N, step=L)
    def _(i):
        agg[pl.ds(i, L)] = jnp.zeros((L,), jnp.float32)

    @pl.loop(0, N_ITER)
    def _(it):
        slot = lax.bitwise_and(it, 1)

        @plsc.parallel_loop(0, EQ, step=L, unroll=8)
        def _(e):
            s = src[pl.ds(e, L)]
            t = dst[pl.ds(e, L)]
            ys = plsc.load_gather(y, [s])
            yt = plsc.load_gather(y, [t])
            d = ys - yt
            gv = d * _rsqrt(d * d + EPS)
            plsc.addupdate_scatter(agg, [s], gv)
            plsc.addupdate_scatter(agg, [t], -gv)

        pltpu.sync_copy(agg, shared.at[slot, lg, pl.ds(q * N, N)])
        plsc.subcore_barrier()
        hp = pltpu.async_copy(shared.at[slot, lg], parts, sem)

        @plsc.parallel_loop(0, N, step=L)
        def _(i):
            s = pl.ds(i, L)
            yi = y[s]
            y[s] = yi - STEP * (yi - xv[s])
            agg[s] = jnp.zeros((L,), jnp.float32)

        hp.wait()

        @plsc.parallel_loop(0, N, step=L)
        def _(i):
            s = pl.ds(i, L)
            a = (parts[pl.ds(i, L)] + parts[pl.ds(N + i, L)]) + (
                parts[pl.ds(2 * N + i, L)] + parts[pl.ds(3 * N + i, L)]
            )
            y[s] = y[s] - (STEP * LAM) * a

    def _vmax(i, m):
        return jnp.maximum(m, y[pl.ds(i * L, L)])

    mv = lax.fori_loop(0, N // L, _vmax, jnp.full((L,), -3.4e38, jnp.float32))
    zmax = jnp.max(mv)

    lane_f = lax.convert_element_type(jax.lax.iota(jnp.int32, L), jnp.float32)
    ones = jnp.full((L,), 1.0, jnp.float32)

    def _hist_narrow(lo0, inv_w):
        @plsc.parallel_loop(0, NB, step=L)
        def _(i):
            parts[pl.ds(i, L)] = jnp.zeros((L,), jnp.float32)
            parts[pl.ds(N + i, L)] = jnp.zeros((L,), jnp.float32)

        scale = inv_w * float(NB)

        @plsc.parallel_loop(0, N, step=L, unroll=2)
        def _(i):
            zi = y[pl.ds(i, L)]
            u = (zi - lo0) * scale
            u = jnp.minimum(jnp.maximum(u, 0.0), float(NB - 1))
            bi = lax.convert_element_type(u, jnp.int32)
            plsc.addupdate_scatter(parts, [bi], zi)
            plsc.addupdate_scatter(parts, [bi + N], ones)

        carry_S = jnp.float32(0.0)
        carry_k = jnp.float32(0.0)
        best = lo0
        bw = 1.0 / scale
        for c in range(NB // L - 1, -1, -1):
            sv = parts[pl.ds(c * L, L)]
            kv = parts[pl.ds(N + c * L, L)]
            ssum = lax.rev(plsc.cumsum(lax.rev(sv, (0,))), (0,)) + carry_S
            ksum = lax.rev(plsc.cumsum(lax.rev(kv, (0,))), (0,)) + carry_k
            tb = lo0 + (float(c * L) + lane_f) * bw
            cond = ssum - ksum * tb - 1.0 >= 0.0
            cand = jnp.where(cond, tb, jnp.float32(-3.4e38))
            best = jnp.maximum(best, jnp.max(cand))
            carry_S = carry_S + jnp.sum(sv)
            carry_k = carry_k + jnp.sum(kv)
        return best

    lo1 = _hist_narrow(zmax - 1.0, 1.0)
    lo2 = _hist_narrow(lo1, float(NB))
    lo3 = _hist_narrow(lo2, float(NB * NB))

    def _support(tau):
        def st(i, c):
            Sv, kv = c
            zi = y[pl.ds(i * L, L)]
            msk = zi > tau
            return (
                Sv + jnp.where(msk, zi, 0.0),
                kv + jnp.where(msk, 1.0, 0.0),
            )

        Sv, kv = lax.fori_loop(
            0,
            N // L,
            st,
            (jnp.zeros((L,), jnp.float32), jnp.zeros((L,), jnp.float32)),
        )
        return jnp.sum(Sv), jnp.sum(kv)

    def _bisect(_i, c):
        lo, hi = c
        mid = 0.5 * (lo + hi)
        S, k = _support(mid)
        pos = S - k * mid - 1.0 >= 0.0
        return (jnp.where(pos, mid, lo), jnp.where(pos, hi, mid))

    lo, _hi = lax.fori_loop(0, _N_BISECT, _bisect, (lo3, lo3 + 1.0 / (NB * NB * NB)))

    def _newton(_i, tau):
        S, k = _support(tau)
        return (S - 1.0) * _recip(jnp.maximum(k, 1.0))

    tau = lax.fori_loop(0, _N_NEWTON, _newton, lo)

    q0 = q * NQ

    @pl.loop(0, NQ, step=L)
    def _(i):
        t1[pl.ds(i, L)] = float(N) * jnp.maximum(y[pl.ds(q0 + i, L)] - tau, 0.0)

    pltpu.sync_copy(t1, out_hbm.at[pl.ds(g * N + q0, NQ)])


@jax.jit
def kernel(x, edge_index):
    se = edge_index.astype(jnp.int32).transpose(0, 2, 1)
    mesh = plsc.VectorSubcoreMesh(core_axis_name="c", subcore_axis_name="s")
    cp = pltpu.CompilerParams()
    if "needs_layout_passes" in pltpu.CompilerParams.__dataclass_fields__:
        cp = dataclasses.replace(cp, needs_layout_passes=False)
    run = pl.kernel(
        _sc_body,
        out_type=jax.ShapeDtypeStruct((B * N,), jnp.float32),
        mesh=mesh,
        scratch_types=[
            pltpu.VMEM((N,), jnp.float32),
            pltpu.VMEM((N,), jnp.float32),
            pltpu.VMEM((N,), jnp.float32),
            pltpu.VMEM((NQ,), jnp.float32),
            pltpu.VMEM((4 * N,), jnp.float32),
            pltpu.VMEM((EQ,), jnp.int32),
            pltpu.VMEM((EQ,), jnp.int32),
            pltpu.VMEM_SHARED((2, 4, 4 * N), jnp.float32),
            pltpu.SemaphoreType.DMA,
        ],
        compiler_params=cp,
    )
    return run(x, se)


# speedup vs baseline: 1.0109x; 1.0109x over previous
import dataclasses
import functools

import jax
import jax.numpy as jnp
from jax import lax
from jax.experimental import pallas as pl
from jax.experimental.pallas import tpu as pltpu
from jax.experimental.pallas import tpu_sc as plsc

B = 8
N = 1024
E = 4096
EQ = E // 4
NQ = N // 4
GAMMA = 1.0
LAM = 1.0
N_ITER = 10
STEP = 0.1
EPS = 1e-6
L = 16
NB = 128

_N_BISECT = 0
_N_NEWTON = 2


def _rsqrt(a):
    i = plsc.bitcast(a, jnp.int32)
    i = jnp.int32(0x5F3759DF) - lax.shift_right_arithmetic(i, 1)
    r = plsc.bitcast(i, jnp.float32)
    h = 0.5 * a
    for _ in range(3):
        r = r * (1.5 - h * r * r)
    return r


def _recip(a):
    av = jnp.full((L,), 1.0, jnp.float32) * a
    i = jnp.int32(0x7EF311C3) - plsc.bitcast(av, jnp.int32)
    r = plsc.bitcast(i, jnp.float32)
    for _ in range(4):
        r = r * (2.0 - av * r)
    return jnp.max(r)


def _sc_body(x_hbm, se_hbm, out_hbm, xv, y, agg, t1, parts, src, dst, shared, sem):
    cid = lax.axis_index("c")
    sid = lax.axis_index("s")
    lg = lax.shift_right_logical(sid, 2)
    q = lax.bitwise_and(sid, 3)
    g = cid * 4 + lg

    hs = [
        pltpu.async_copy(x_hbm.at[pl.ds(g * N, N)], xv, sem),
        pltpu.async_copy(x_hbm.at[pl.ds(g * N, N)], y, sem),
        pltpu.async_copy(se_hbm.at[g, 0, pl.ds(q * EQ, EQ)], src, sem),
        pltpu.async_copy(se_hbm.at[g, 1, pl.ds(q * EQ, EQ)], dst, sem),
    ]
    for h in hs:
        h.wait()

    @plsc.parallel_loop(0, N, step=L)
    def _(i):
        agg[pl.ds(i, L)] = jnp.zeros((L,), jnp.float32)

    @pl.loop(0, N_ITER)
    def _(it):
        slot = lax.bitwise_and(it, 1)

        @plsc.parallel_loop(0, EQ, step=L, unroll=4)
        def _(e):
            s = src[pl.ds(e, L)]
            t = dst[pl.ds(e, L)]
            ys = plsc.load_gather(y, [s])
            yt = plsc.load_gather(y, [t])
            d = ys - yt
            gv = d * _rsqrt(d * d + EPS)
            plsc.addupdate_scatter(agg, [s], gv)
            plsc.addupdate_scatter(agg, [t], -gv)

        pltpu.sync_copy(agg, shared.at[slot, lg, pl.ds(q * N, N)])
        plsc.subcore_barrier()
        hp = pltpu.async_copy(shared.at[slot, lg], parts, sem)

        @plsc.parallel_loop(0, N, step=L)
        def _(i):
            s = pl.ds(i, L)
            yi = y[s]
            y[s] = yi - STEP * (yi - xv[s])
            agg[s] = jnp.zeros((L,), jnp.float32)

        hp.wait()

        @plsc.parallel_loop(0, N, step=L)
        def _(i):
            s = pl.ds(i, L)
            a = (parts[pl.ds(i, L)] + parts[pl.ds(N + i, L)]) + (
                parts[pl.ds(2 * N + i, L)] + parts[pl.ds(3 * N + i, L)]
            )
            y[s] = y[s] - (STEP * LAM) * a

    def _vmax(i, m):
        return jnp.maximum(m, y[pl.ds(i * L, L)])

    mv = lax.fori_loop(0, N // L, _vmax, jnp.full((L,), -3.4e38, jnp.float32))
    zmax = jnp.max(mv)

    lane_f = lax.convert_element_type(jax.lax.iota(jnp.int32, L), jnp.float32)
    ones = jnp.full((L,), 1.0, jnp.float32)

    def _hist_narrow(lo0, inv_w):
        @plsc.parallel_loop(0, NB, step=L)
        def _(i):
            parts[pl.ds(i, L)] = jnp.zeros((L,), jnp.float32)
            parts[pl.ds(N + i, L)] = jnp.zeros((L,), jnp.float32)

        scale = inv_w * float(NB)

        @plsc.parallel_loop(0, N, step=L, unroll=2)
        def _(i):
            zi = y[pl.ds(i, L)]
            u = (zi - lo0) * scale
            u = jnp.minimum(jnp.maximum(u, 0.0), float(NB - 1))
            bi = lax.convert_element_type(u, jnp.int32)
            plsc.addupdate_scatter(parts, [bi], zi)
            plsc.addupdate_scatter(parts, [bi + N], ones)

        carry_S = jnp.float32(0.0)
        carry_k = jnp.float32(0.0)
        best = lo0
        bw = 1.0 / scale
        for c in range(NB // L - 1, -1, -1):
            sv = parts[pl.ds(c * L, L)]
            kv = parts[pl.ds(N + c * L, L)]
            ssum = lax.rev(plsc.cumsum(lax.rev(sv, (0,))), (0,)) + carry_S
            ksum = lax.rev(plsc.cumsum(lax.rev(kv, (0,))), (0,)) + carry_k
            tb = lo0 + (float(c * L) + lane_f) * bw
            cond = ssum - ksum * tb - 1.0 >= 0.0
            cand = jnp.where(cond, tb, jnp.float32(-3.4e38))
            best = jnp.maximum(best, jnp.max(cand))
            carry_S = carry_S + jnp.sum(sv)
            carry_k = carry_k + jnp.sum(kv)
        return best

    lo1 = _hist_narrow(zmax - 1.0, 1.0)
    lo2 = _hist_narrow(lo1, float(NB))
    lo3 = _hist_narrow(lo2, float(NB * NB))

    def _support(tau):
        def st(i, c):
            Sv, kv = c
            zi = y[pl.ds(i * L, L)]
            msk = zi > tau
            return (
                Sv + jnp.where(msk, zi, 0.0),
                kv + jnp.where(msk, 1.0, 0.0),
            )

        Sv, kv = lax.fori_loop(
            0,
            N // L,
            st,
            (jnp.zeros((L,), jnp.float32), jnp.zeros((L,), jnp.float32)),
        )
        return jnp.sum(Sv), jnp.sum(kv)

    def _bisect(_i, c):
        lo, hi = c
        mid = 0.5 * (lo + hi)
        S, k = _support(mid)
        pos = S - k * mid - 1.0 >= 0.0
        return (jnp.where(pos, mid, lo), jnp.where(pos, hi, mid))

    lo, _hi = lax.fori_loop(0, _N_BISECT, _bisect, (lo3, lo3 + 1.0 / (NB * NB * NB)))

    def _newton(_i, tau):
        S, k = _support(tau)
        return (S - 1.0) * _recip(jnp.maximum(k, 1.0))

    tau = lax.fori_loop(0, _N_NEWTON, _newton, lo)

    q0 = q * NQ

    @pl.loop(0, NQ, step=L)
    def _(i):
        t1[pl.ds(i, L)] = float(N) * jnp.maximum(y[pl.ds(q0 + i, L)] - tau, 0.0)

    pltpu.sync_copy(t1, out_hbm.at[pl.ds(g * N + q0, NQ)])


@jax.jit
def kernel(x, edge_index):
    se = edge_index.astype(jnp.int32).transpose(0, 2, 1)
    mesh = plsc.VectorSubcoreMesh(core_axis_name="c", subcore_axis_name="s")
    cp = pltpu.CompilerParams()
    if "needs_layout_passes" in pltpu.CompilerParams.__dataclass_fields__:
        cp = dataclasses.replace(cp, needs_layout_passes=False)
    run = pl.kernel(
        _sc_body,
        out_type=jax.ShapeDtypeStruct((B * N,), jnp.float32),
        mesh=mesh,
        scratch_types=[
            pltpu.VMEM((N,), jnp.float32),
            pltpu.VMEM((N,), jnp.float32),
            pltpu.VMEM((N,), jnp.float32),
            pltpu.VMEM((NQ,), jnp.float32),
            pltpu.VMEM((4 * N,), jnp.float32),
            pltpu.VMEM((EQ,), jnp.int32),
            pltpu.VMEM((EQ,), jnp.int32),
            pltpu.VMEM_SHARED((2, 4, 4 * N), jnp.float32),
            pltpu.SemaphoreType.DMA,
        ],
        compiler_params=cp,
    )
    return run(x, se)


# speedup vs baseline: 1.1021x; 1.0902x over previous
import dataclasses
import functools

import jax
import jax.numpy as jnp
from jax import lax
from jax.experimental import pallas as pl
from jax.experimental.pallas import tpu as pltpu
from jax.experimental.pallas import tpu_sc as plsc

B = 8
N = 1024
E = 4096
EQ = E // 4
NQ = N // 4
GAMMA = 1.0
LAM = 1.0
N_ITER = 10
STEP = 0.1
EPS = 1e-6
L = 16
NB = 128

_N_BISECT = 11
_N_NEWTON = 2


def _rsqrt(a):
    i = plsc.bitcast(a, jnp.int32)
    i = jnp.int32(0x5F3759DF) - lax.shift_right_arithmetic(i, 1)
    r = plsc.bitcast(i, jnp.float32)
    h = 0.5 * a
    for _ in range(3):
        r = r * (1.5 - h * r * r)
    return r


def _recip(a):
    av = jnp.full((L,), 1.0, jnp.float32) * a
    i = jnp.int32(0x7EF311C3) - plsc.bitcast(av, jnp.int32)
    r = plsc.bitcast(i, jnp.float32)
    for _ in range(4):
        r = r * (2.0 - av * r)
    return jnp.max(r)


def _sc_body(x_hbm, se_hbm, out_hbm, xv, y, agg, t1, parts, src, dst, shared, sem):
    cid = lax.axis_index("c")
    sid = lax.axis_index("s")
    lg = lax.shift_right_logical(sid, 2)
    q = lax.bitwise_and(sid, 3)
    g = cid * 4 + lg

    hs = [
        pltpu.async_copy(x_hbm.at[pl.ds(g * N, N)], xv, sem),
        pltpu.async_copy(x_hbm.at[pl.ds(g * N, N)], y, sem),
        pltpu.async_copy(se_hbm.at[g, 0, pl.ds(q * EQ, EQ)], src, sem),
        pltpu.async_copy(se_hbm.at[g, 1, pl.ds(q * EQ, EQ)], dst, sem),
    ]
    for h in hs:
        h.wait()

    @plsc.parallel_loop(0, N, step=L)
    def _(i):
        agg[pl.ds(i, L)] = jnp.zeros((L,), jnp.float32)

    @pl.loop(0, N_ITER)
    def _(it):
        slot = lax.bitwise_and(it, 1)

        @plsc.parallel_loop(0, EQ, step=L, unroll=4)
        def _(e):
            s = src[pl.ds(e, L)]
            t = dst[pl.ds(e, L)]
            ys = plsc.load_gather(y, [s])
            yt = plsc.load_gather(y, [t])
            d = ys - yt
            gv = d * _rsqrt(d * d + EPS)
            plsc.addupdate_scatter(agg, [s], gv)
            plsc.addupdate_scatter(agg, [t], -gv)

        pltpu.sync_copy(agg, shared.at[slot, lg, pl.ds(q * N, N)])
        plsc.subcore_barrier()
        hp = pltpu.async_copy(shared.at[slot, lg], parts, sem)

        @plsc.parallel_loop(0, N, step=L)
        def _(i):
            s = pl.ds(i, L)
            yi = y[s]
            y[s] = yi - STEP * (yi - xv[s])
            agg[s] = jnp.zeros((L,), jnp.float32)

        hp.wait()

        @plsc.parallel_loop(0, N, step=L)
        def _(i):
            s = pl.ds(i, L)
            a = (parts[pl.ds(i, L)] + parts[pl.ds(N + i, L)]) + (
                parts[pl.ds(2 * N + i, L)] + parts[pl.ds(3 * N + i, L)]
            )
            y[s] = y[s] - (STEP * LAM) * a

    def _vmax(i, m):
        return jnp.maximum(m, y[pl.ds(i * L, L)])

    mv = lax.fori_loop(0, N // L, _vmax, jnp.full((L,), -3.4e38, jnp.float32))
    zmax = jnp.max(mv)

    lane_f = lax.convert_element_type(jax.lax.iota(jnp.int32, L), jnp.float32)
    ones = jnp.full((L,), 1.0, jnp.float32)

    def _hist_narrow(lo0, inv_w):
        @plsc.parallel_loop(0, NB, step=L)
        def _(i):
            parts[pl.ds(i, L)] = jnp.zeros((L,), jnp.float32)
            parts[pl.ds(N + i, L)] = jnp.zeros((L,), jnp.float32)

        scale = inv_w * float(NB)

        @plsc.parallel_loop(0, N, step=L, unroll=2)
        def _(i):
            zi = y[pl.ds(i, L)]
            u = (zi - lo0) * scale
            u = jnp.minimum(jnp.maximum(u, 0.0), float(NB - 1))
            bi = lax.convert_element_type(u, jnp.int32)
            plsc.addupdate_scatter(parts, [bi], zi)
            plsc.addupdate_scatter(parts, [bi + N], ones)

        carry_S = jnp.float32(0.0)
        carry_k = jnp.float32(0.0)
        best = lo0
        bw = 1.0 / scale
        for c in range(NB // L - 1, -1, -1):
            sv = parts[pl.ds(c * L, L)]
            kv = parts[pl.ds(N + c * L, L)]
            ssum = lax.rev(plsc.cumsum(lax.rev(sv, (0,))), (0,)) + carry_S
            ksum = lax.rev(plsc.cumsum(lax.rev(kv, (0,))), (0,)) + carry_k
            tb = lo0 + (float(c * L) + lane_f) * bw
            cond = ssum - ksum * tb - 1.0 >= 0.0
            cand = jnp.where(cond, tb, jnp.float32(-3.4e38))
            best = jnp.maximum(best, jnp.max(cand))
            carry_S = carry_S + jnp.sum(sv)
            carry_k = carry_k + jnp.sum(kv)
        return best

    lo2 = _hist_narrow(zmax - 1.0, 1.0)

    def _support(tau):
        def st(i, c):
            Sv, kv = c
            zi = y[pl.ds(i * L, L)]
            msk = zi > tau
            return (
                Sv + jnp.where(msk, zi, 0.0),
                kv + jnp.where(msk, 1.0, 0.0),
            )

        Sv, kv = lax.fori_loop(
            0,
            N // L,
            st,
            (jnp.zeros((L,), jnp.float32), jnp.zeros((L,), jnp.float32)),
        )
        return jnp.sum(Sv), jnp.sum(kv)

    def _bisect(_i, c):
        lo, hi = c
        mid = 0.5 * (lo + hi)
        S, k = _support(mid)
        pos = S - k * mid - 1.0 >= 0.0
        return (jnp.where(pos, mid, lo), jnp.where(pos, hi, mid))

    lo, _hi = lax.fori_loop(0, _N_BISECT, _bisect, (lo2, lo2 + 1.0 / NB))

    def _newton(_i, tau):
        S, k = _support(tau)
        return (S - 1.0) * _recip(jnp.maximum(k, 1.0))

    tau = lax.fori_loop(0, _N_NEWTON, _newton, lo)

    q0 = q * NQ

    @pl.loop(0, NQ, step=L)
    def _(i):
        t1[pl.ds(i, L)] = float(N) * jnp.maximum(y[pl.ds(q0 + i, L)] - tau, 0.0)

    pltpu.sync_copy(t1, out_hbm.at[pl.ds(g * N + q0, NQ)])


@jax.jit
def kernel(x, edge_index):
    se = edge_index.astype(jnp.int32).transpose(0, 2, 1)
    mesh = plsc.VectorSubcoreMesh(core_axis_name="c", subcore_axis_name="s")
    cp = pltpu.CompilerParams()
    if "needs_layout_passes" in pltpu.CompilerParams.__dataclass_fields__:
        cp = dataclasses.replace(cp, needs_layout_passes=False)
    run = pl.kernel(
        _sc_body,
        out_type=jax.ShapeDtypeStruct((B * N,), jnp.float32),
        mesh=mesh,
        scratch_types=[
            pltpu.VMEM((N,), jnp.float32),
            pltpu.VMEM((N,), jnp.float32),
            pltpu.VMEM((N,), jnp.float32),
            pltpu.VMEM((NQ,), jnp.float32),
            pltpu.VMEM((4 * N,), jnp.float32),
            pltpu.VMEM((EQ,), jnp.int32),
            pltpu.VMEM((EQ,), jnp.int32),
            pltpu.VMEM_SHARED((2, 4, 4 * N), jnp.float32),
            pltpu.SemaphoreType.DMA,
        ],
        compiler_params=cp,
    )
    return run(x, se)


# speedup vs baseline: 1.1255x; 1.0213x over previous
import dataclasses
import functools

import jax
import jax.numpy as jnp
from jax import lax
from jax.experimental import pallas as pl
from jax.experimental.pallas import tpu as pltpu
from jax.experimental.pallas import tpu_sc as plsc

B = 8
N = 1024
E = 4096
EQ = E // 4
NQ = N // 4
GAMMA = 1.0
LAM = 1.0
N_ITER = 10
STEP = 0.1
EPS = 1e-6
L = 16
NB = 128

_N_BISECT = 19
_N_NEWTON = 2


def _rsqrt(a):
    i = plsc.bitcast(a, jnp.int32)
    i = jnp.int32(0x5F3759DF) - lax.shift_right_arithmetic(i, 1)
    r = plsc.bitcast(i, jnp.float32)
    h = 0.5 * a
    for _ in range(3):
        r = r * (1.5 - h * r * r)
    return r


def _recip(a):
    av = jnp.full((L,), 1.0, jnp.float32) * a
    i = jnp.int32(0x7EF311C3) - plsc.bitcast(av, jnp.int32)
    r = plsc.bitcast(i, jnp.float32)
    for _ in range(4):
        r = r * (2.0 - av * r)
    return jnp.max(r)


def _sc_body(x_hbm, se_hbm, out_hbm, xv, y, agg, t1, parts, src, dst, shared, sem):
    cid = lax.axis_index("c")
    sid = lax.axis_index("s")
    lg = lax.shift_right_logical(sid, 2)
    q = lax.bitwise_and(sid, 3)
    g = cid * 4 + lg

    hs = [
        pltpu.async_copy(x_hbm.at[pl.ds(g * N, N)], xv, sem),
        pltpu.async_copy(x_hbm.at[pl.ds(g * N, N)], y, sem),
        pltpu.async_copy(se_hbm.at[g, 0, pl.ds(q * EQ, EQ)], src, sem),
        pltpu.async_copy(se_hbm.at[g, 1, pl.ds(q * EQ, EQ)], dst, sem),
    ]
    for h in hs:
        h.wait()

    @plsc.parallel_loop(0, N, step=L)
    def _(i):
        agg[pl.ds(i, L)] = jnp.zeros((L,), jnp.float32)

    @pl.loop(0, N_ITER)
    def _(it):
        slot = lax.bitwise_and(it, 1)

        @plsc.parallel_loop(0, EQ, step=L, unroll=4)
        def _(e):
            s = src[pl.ds(e, L)]
            t = dst[pl.ds(e, L)]
            ys = plsc.load_gather(y, [s])
            yt = plsc.load_gather(y, [t])
            d = ys - yt
            gv = d * _rsqrt(d * d + EPS)
            plsc.addupdate_scatter(agg, [s], gv)
            plsc.addupdate_scatter(agg, [t], -gv)

        pltpu.sync_copy(agg, shared.at[slot, lg, pl.ds(q * N, N)])
        plsc.subcore_barrier()
        hp = pltpu.async_copy(shared.at[slot, lg], parts, sem)

        @plsc.parallel_loop(0, N, step=L)
        def _(i):
            s = pl.ds(i, L)
            yi = y[s]
            y[s] = yi - STEP * (yi - xv[s])
            agg[s] = jnp.zeros((L,), jnp.float32)

        hp.wait()

        @plsc.parallel_loop(0, N, step=L)
        def _(i):
            s = pl.ds(i, L)
            a = (parts[pl.ds(i, L)] + parts[pl.ds(N + i, L)]) + (
                parts[pl.ds(2 * N + i, L)] + parts[pl.ds(3 * N + i, L)]
            )
            y[s] = y[s] - (STEP * LAM) * a

    def _vmax(i, m):
        return jnp.maximum(m, y[pl.ds(i * L, L)])

    mv = lax.fori_loop(0, N // L, _vmax, jnp.full((L,), -3.4e38, jnp.float32))
    zmax = jnp.max(mv)

    lane_f = lax.convert_element_type(jax.lax.iota(jnp.int32, L), jnp.float32)
    ones = jnp.full((L,), 1.0, jnp.float32)

    def _hist_narrow(lo0, inv_w):
        @plsc.parallel_loop(0, NB, step=L)
        def _(i):
            parts[pl.ds(i, L)] = jnp.zeros((L,), jnp.float32)
            parts[pl.ds(N + i, L)] = jnp.zeros((L,), jnp.float32)

        scale = inv_w * float(NB)

        @plsc.parallel_loop(0, N, step=L, unroll=2)
        def _(i):
            zi = y[pl.ds(i, L)]
            u = (zi - lo0) * scale
            u = jnp.minimum(jnp.maximum(u, 0.0), float(NB - 1))
            bi = lax.convert_element_type(u, jnp.int32)
            plsc.addupdate_scatter(parts, [bi], zi)
            plsc.addupdate_scatter(parts, [bi + N], ones)

        carry_S = jnp.float32(0.0)
        carry_k = jnp.float32(0.0)
        best = lo0
        bw = 1.0 / scale
        for c in range(NB // L - 1, -1, -1):
            sv = parts[pl.ds(c * L, L)]
            kv = parts[pl.ds(N + c * L, L)]
            ssum = lax.rev(plsc.cumsum(lax.rev(sv, (0,))), (0,)) + carry_S
            ksum = lax.rev(plsc.cumsum(lax.rev(kv, (0,))), (0,)) + carry_k
            tb = lo0 + (float(c * L) + lane_f) * bw
            cond = ssum - ksum * tb - 1.0 >= 0.0
            cand = jnp.where(cond, tb, jnp.float32(-3.4e38))
            best = jnp.maximum(best, jnp.max(cand))
            carry_S = carry_S + jnp.sum(sv)
            carry_k = carry_k + jnp.sum(kv)
        return best

    lo2 = zmax - 1.0

    def _support(tau):
        def st(i, c):
            Sv, kv = c
            zi = y[pl.ds(i * L, L)]
            msk = zi > tau
            return (
                Sv + jnp.where(msk, zi, 0.0),
                kv + jnp.where(msk, 1.0, 0.0),
            )

        Sv, kv = lax.fori_loop(
            0,
            N // L,
            st,
            (jnp.zeros((L,), jnp.float32), jnp.zeros((L,), jnp.float32)),
        )
        return jnp.sum(Sv), jnp.sum(kv)

    def _bisect(_i, c):
        lo, hi = c
        mid = 0.5 * (lo + hi)
        S, k = _support(mid)
        pos = S - k * mid - 1.0 >= 0.0
        return (jnp.where(pos, mid, lo), jnp.where(pos, hi, mid))

    lo, _hi = lax.fori_loop(0, _N_BISECT, _bisect, (lo2, lo2 + 1.0))

    def _newton(_i, tau):
        S, k = _support(tau)
        return (S - 1.0) * _recip(jnp.maximum(k, 1.0))

    tau = lax.fori_loop(0, _N_NEWTON, _newton, lo)

    q0 = q * NQ

    @pl.loop(0, NQ, step=L)
    def _(i):
        t1[pl.ds(i, L)] = float(N) * jnp.maximum(y[pl.ds(q0 + i, L)] - tau, 0.0)

    pltpu.sync_copy(t1, out_hbm.at[pl.ds(g * N + q0, NQ)])


@jax.jit
def kernel(x, edge_index):
    se = edge_index.astype(jnp.int32).transpose(0, 2, 1)
    mesh = plsc.VectorSubcoreMesh(core_axis_name="c", subcore_axis_name="s")
    cp = pltpu.CompilerParams()
    if "needs_layout_passes" in pltpu.CompilerParams.__dataclass_fields__:
        cp = dataclasses.replace(cp, needs_layout_passes=False)
    run = pl.kernel(
        _sc_body,
        out_type=jax.ShapeDtypeStruct((B * N,), jnp.float32),
        mesh=mesh,
        scratch_types=[
            pltpu.VMEM((N,), jnp.float32),
            pltpu.VMEM((N,), jnp.float32),
            pltpu.VMEM((N,), jnp.float32),
            pltpu.VMEM((NQ,), jnp.float32),
            pltpu.VMEM((4 * N,), jnp.float32),
            pltpu.VMEM((EQ,), jnp.int32),
            pltpu.VMEM((EQ,), jnp.int32),
            pltpu.VMEM_SHARED((2, 4, 4 * N), jnp.float32),
            pltpu.SemaphoreType.DMA,
        ],
        compiler_params=cp,
    )
    return run(x, se)


# speedup vs baseline: 1.2232x; 1.0867x over previous
import dataclasses
import functools

import jax
import jax.numpy as jnp
from jax import lax
from jax.experimental import pallas as pl
from jax.experimental.pallas import tpu as pltpu
from jax.experimental.pallas import tpu_sc as plsc

B = 8
N = 1024
E = 4096
EQ = E // 4
NQ = N // 4
GAMMA = 1.0
LAM = 1.0
N_ITER = 10
STEP = 0.1
EPS = 1e-6
L = 16
NB = 128

_N_BISECT = 13
_N_NEWTON = 2


def _rsqrt(a):
    i = plsc.bitcast(a, jnp.int32)
    i = jnp.int32(0x5F3759DF) - lax.shift_right_arithmetic(i, 1)
    r = plsc.bitcast(i, jnp.float32)
    h = 0.5 * a
    for _ in range(3):
        r = r * (1.5 - h * r * r)
    return r


def _recip(a):
    av = jnp.full((L,), 1.0, jnp.float32) * a
    i = jnp.int32(0x7EF311C3) - plsc.bitcast(av, jnp.int32)
    r = plsc.bitcast(i, jnp.float32)
    for _ in range(4):
        r = r * (2.0 - av * r)
    return jnp.max(r)


def _sc_body(x_hbm, se_hbm, out_hbm, xv, y, agg, t1, parts, src, dst, shared, sem):
    cid = lax.axis_index("c")
    sid = lax.axis_index("s")
    lg = lax.shift_right_logical(sid, 2)
    q = lax.bitwise_and(sid, 3)
    g = cid * 4 + lg

    hs = [
        pltpu.async_copy(x_hbm.at[pl.ds(g * N, N)], xv, sem),
        pltpu.async_copy(x_hbm.at[pl.ds(g * N, N)], y, sem),
        pltpu.async_copy(se_hbm.at[g, 0, pl.ds(q * EQ, EQ)], src, sem),
        pltpu.async_copy(se_hbm.at[g, 1, pl.ds(q * EQ, EQ)], dst, sem),
    ]
    for h in hs:
        h.wait()

    @plsc.parallel_loop(0, N, step=L)
    def _(i):
        agg[pl.ds(i, L)] = jnp.zeros((L,), jnp.float32)

    @pl.loop(0, N_ITER)
    def _(it):
        slot = lax.bitwise_and(it, 1)

        @plsc.parallel_loop(0, EQ, step=L, unroll=4)
        def _(e):
            s = src[pl.ds(e, L)]
            t = dst[pl.ds(e, L)]
            ys = plsc.load_gather(y, [s])
            yt = plsc.load_gather(y, [t])
            d = ys - yt
            gv = d * _rsqrt(d * d + EPS)
            plsc.addupdate_scatter(agg, [s], gv)
            plsc.addupdate_scatter(agg, [t], -gv)

        pltpu.sync_copy(agg, shared.at[slot, lg, pl.ds(q * N, N)])
        plsc.subcore_barrier()
        hp = pltpu.async_copy(shared.at[slot, lg], parts, sem)

        @plsc.parallel_loop(0, N, step=L)
        def _(i):
            s = pl.ds(i, L)
            yi = y[s]
            y[s] = yi - STEP * (yi - xv[s])
            agg[s] = jnp.zeros((L,), jnp.float32)

        hp.wait()

        @plsc.parallel_loop(0, N, step=L)
        def _(i):
            s = pl.ds(i, L)
            a = (parts[pl.ds(i, L)] + parts[pl.ds(N + i, L)]) + (
                parts[pl.ds(2 * N + i, L)] + parts[pl.ds(3 * N + i, L)]
            )
            y[s] = y[s] - (STEP * LAM) * a

    @plsc.parallel_loop(0, N, step=L, unroll=2,
                        carry=jnp.full((L,), -3.4e38, jnp.float32))
    def _vmax(i, m):
        return jnp.maximum(m, y[pl.ds(i, L)])

    zmax = jnp.max(_vmax)

    lane_f = lax.convert_element_type(jax.lax.iota(jnp.int32, L), jnp.float32)
    ones = jnp.full((L,), 1.0, jnp.float32)

    def _hist_narrow(lo0, inv_w):
        @plsc.parallel_loop(0, NB, step=L)
        def _(i):
            parts[pl.ds(i, L)] = jnp.zeros((L,), jnp.float32)
            parts[pl.ds(N + i, L)] = jnp.zeros((L,), jnp.float32)

        scale = inv_w * float(NB)

        @plsc.parallel_loop(0, N, step=L, unroll=2)
        def _(i):
            zi = y[pl.ds(i, L)]
            u = (zi - lo0) * scale
            u = jnp.minimum(jnp.maximum(u, 0.0), float(NB - 1))
            bi = lax.convert_element_type(u, jnp.int32)
            plsc.addupdate_scatter(parts, [bi], zi)
            plsc.addupdate_scatter(parts, [bi + N], ones)

        carry_S = jnp.float32(0.0)
        carry_k = jnp.float32(0.0)
        best = lo0
        bw = 1.0 / scale
        for c in range(NB // L - 1, -1, -1):
            sv = parts[pl.ds(c * L, L)]
            kv = parts[pl.ds(N + c * L, L)]
            ssum = lax.rev(plsc.cumsum(lax.rev(sv, (0,))), (0,)) + carry_S
            ksum = lax.rev(plsc.cumsum(lax.rev(kv, (0,))), (0,)) + carry_k
            tb = lo0 + (float(c * L) + lane_f) * bw
            cond = ssum - ksum * tb - 1.0 >= 0.0
            cand = jnp.where(cond, tb, jnp.float32(-3.4e38))
            best = jnp.maximum(best, jnp.max(cand))
            carry_S = carry_S + jnp.sum(sv)
            carry_k = carry_k + jnp.sum(kv)
        return best

    lo2 = zmax - 1.0

    def _support(tau):
        def st(i, c):
            Sv, kv = c
            zi = y[pl.ds(i, L)]
            msk = zi > tau
            return (
                Sv + jnp.where(msk, zi, 0.0),
                kv + jnp.where(msk, 1.0, 0.0),
            )

        Sv, kv = plsc.parallel_loop(
            0, N, step=L, unroll=2,
            carry=(jnp.zeros((L,), jnp.float32), jnp.zeros((L,), jnp.float32)),
        )(st)
        return jnp.sum(Sv), jnp.sum(kv)

    def _bisect(_i, c):
        lo, hi = c
        mid = 0.5 * (lo + hi)
        S, k = _support(mid)
        pos = S - k * mid - 1.0 >= 0.0
        return (jnp.where(pos, mid, lo), jnp.where(pos, hi, mid))

    lo, _hi = lax.fori_loop(0, _N_BISECT, _bisect, (lo2, lo2 + 1.0))

    def _newton(_i, tau):
        S, k = _support(tau)
        return (S - 1.0) * _recip(jnp.maximum(k, 1.0))

    tau = lax.fori_loop(0, _N_NEWTON, _newton, lo)

    q0 = q * NQ

    @pl.loop(0, NQ, step=L)
    def _(i):
        t1[pl.ds(i, L)] = float(N) * jnp.maximum(y[pl.ds(q0 + i, L)] - tau, 0.0)

    pltpu.sync_copy(t1, out_hbm.at[pl.ds(g * N + q0, NQ)])


@jax.jit
def kernel(x, edge_index):
    se = edge_index.astype(jnp.int32).transpose(0, 2, 1)
    mesh = plsc.VectorSubcoreMesh(core_axis_name="c", subcore_axis_name="s")
    cp = pltpu.CompilerParams()
    if "needs_layout_passes" in pltpu.CompilerParams.__dataclass_fields__:
        cp = dataclasses.replace(cp, needs_layout_passes=False)
    run = pl.kernel(
        _sc_body,
        out_type=jax.ShapeDtypeStruct((B * N,), jnp.float32),
        mesh=mesh,
        scratch_types=[
            pltpu.VMEM((N,), jnp.float32),
            pltpu.VMEM((N,), jnp.float32),
            pltpu.VMEM((N,), jnp.float32),
            pltpu.VMEM((NQ,), jnp.float32),
            pltpu.VMEM((4 * N,), jnp.float32),
            pltpu.VMEM((EQ,), jnp.int32),
            pltpu.VMEM((EQ,), jnp.int32),
            pltpu.VMEM_SHARED((2, 4, 4 * N), jnp.float32),
            pltpu.SemaphoreType.DMA,
        ],
        compiler_params=cp,
    )
    return run(x, se)


# speedup vs baseline: 1.2256x; 1.0020x over previous
import dataclasses
import functools

import jax
import jax.numpy as jnp
from jax import lax
from jax.experimental import pallas as pl
from jax.experimental.pallas import tpu as pltpu
from jax.experimental.pallas import tpu_sc as plsc

B = 8
N = 1024
E = 4096
EQ = E // 4
NQ = N // 4
GAMMA = 1.0
LAM = 1.0
N_ITER = 10
STEP = 0.1
EPS = 1e-6
L = 16
NB = 128

_N_BISECT = 13
_N_NEWTON = 2


def _rsqrt(a):
    i = plsc.bitcast(a, jnp.int32)
    i = jnp.int32(0x5F3759DF) - lax.shift_right_arithmetic(i, 1)
    r = plsc.bitcast(i, jnp.float32)
    h = 0.5 * a
    for _ in range(3):
        r = r * (1.5 - h * r * r)
    return r


def _recip(a):
    av = jnp.full((L,), 1.0, jnp.float32) * a
    i = jnp.int32(0x7EF311C3) - plsc.bitcast(av, jnp.int32)
    r = plsc.bitcast(i, jnp.float32)
    for _ in range(4):
        r = r * (2.0 - av * r)
    return jnp.max(r)


def _sc_body(x_hbm, se_hbm, out_hbm, xv, y, agg, t1, parts, src, dst, shared, sem):
    cid = lax.axis_index("c")
    sid = lax.axis_index("s")
    lg = lax.shift_right_logical(sid, 2)
    q = lax.bitwise_and(sid, 3)
    g = cid * 4 + lg

    hs = [
        pltpu.async_copy(x_hbm.at[pl.ds(g * N, N)], xv, sem),
        pltpu.async_copy(x_hbm.at[pl.ds(g * N, N)], y, sem),
        pltpu.async_copy(se_hbm.at[g, 0, pl.ds(q * EQ, EQ)], src, sem),
        pltpu.async_copy(se_hbm.at[g, 1, pl.ds(q * EQ, EQ)], dst, sem),
    ]
    for h in hs:
        h.wait()

    @plsc.parallel_loop(0, N, step=L)
    def _(i):
        agg[pl.ds(i, L)] = jnp.zeros((L,), jnp.float32)

    @pl.loop(0, N_ITER)
    def _(it):
        slot = lax.bitwise_and(it, 1)

        @plsc.parallel_loop(0, EQ, step=L, unroll=4)
        def _(e):
            s = src[pl.ds(e, L)]
            t = dst[pl.ds(e, L)]
            ys = plsc.load_gather(y, [s])
            yt = plsc.load_gather(y, [t])
            d = ys - yt
            gv = d * _rsqrt(d * d + EPS)
            plsc.addupdate_scatter(agg, [s], gv)
            plsc.addupdate_scatter(agg, [t], -gv)

        pltpu.sync_copy(agg, shared.at[slot, lg, pl.ds(q * N, N)])
        plsc.subcore_barrier()
        hp = pltpu.async_copy(shared.at[slot, lg], parts, sem)

        @plsc.parallel_loop(0, N, step=L)
        def _(i):
            s = pl.ds(i, L)
            yi = y[s]
            y[s] = yi - STEP * (yi - xv[s])
            agg[s] = jnp.zeros((L,), jnp.float32)

        hp.wait()

        @plsc.parallel_loop(0, N, step=L)
        def _(i):
            s = pl.ds(i, L)
            a = (parts[pl.ds(i, L)] + parts[pl.ds(N + i, L)]) + (
                parts[pl.ds(2 * N + i, L)] + parts[pl.ds(3 * N + i, L)]
            )
            y[s] = y[s] - (STEP * LAM) * a

    @plsc.parallel_loop(0, N, step=L, unroll=2,
                        carry=jnp.full((L,), -3.4e38, jnp.float32))
    def _vmax(i, m):
        return jnp.maximum(m, y[pl.ds(i, L)])

    zmax = jnp.max(_vmax)

    lane_f = lax.convert_element_type(jax.lax.iota(jnp.int32, L), jnp.float32)
    ones = jnp.full((L,), 1.0, jnp.float32)

    def _hist_narrow(lo0, inv_w):
        @plsc.parallel_loop(0, NB, step=L)
        def _(i):
            parts[pl.ds(i, L)] = jnp.zeros((L,), jnp.float32)
            parts[pl.ds(N + i, L)] = jnp.zeros((L,), jnp.float32)

        scale = inv_w * float(NB)

        @plsc.parallel_loop(0, N, step=L, unroll=2)
        def _(i):
            zi = y[pl.ds(i, L)]
            u = (zi - lo0) * scale
            u = jnp.minimum(jnp.maximum(u, 0.0), float(NB - 1))
            bi = lax.convert_element_type(u, jnp.int32)
            plsc.addupdate_scatter(parts, [bi], zi)
            plsc.addupdate_scatter(parts, [bi + N], ones)

        carry_S = jnp.float32(0.0)
        carry_k = jnp.float32(0.0)
        best = lo0
        bw = 1.0 / scale
        for c in range(NB // L - 1, -1, -1):
            sv = parts[pl.ds(c * L, L)]
            kv = parts[pl.ds(N + c * L, L)]
            ssum = lax.rev(plsc.cumsum(lax.rev(sv, (0,))), (0,)) + carry_S
            ksum = lax.rev(plsc.cumsum(lax.rev(kv, (0,))), (0,)) + carry_k
            tb = lo0 + (float(c * L) + lane_f) * bw
            cond = ssum - ksum * tb - 1.0 >= 0.0
            cand = jnp.where(cond, tb, jnp.float32(-3.4e38))
            best = jnp.maximum(best, jnp.max(cand))
            carry_S = carry_S + jnp.sum(sv)
            carry_k = carry_k + jnp.sum(kv)
        return best

    lo2 = zmax - 1.0

    def _support(tau):
        def st(i, c):
            Sv, kv = c
            zi = y[pl.ds(i, L)]
            msk = zi > tau
            return (
                Sv + jnp.where(msk, zi, 0.0),
                kv + jnp.where(msk, 1.0, 0.0),
            )

        Sv, kv = plsc.parallel_loop(
            0, N, step=L, unroll=2,
            carry=(jnp.zeros((L,), jnp.float32), jnp.zeros((L,), jnp.float32)),
        )(st)
        return jnp.sum(Sv), jnp.sum(kv)

    def _bisect(_i, c):
        lo, hi = c
        mid = 0.5 * (lo + hi)

        def fb(i, fv):
            zi = y[pl.ds(i, L)]
            return fv + jnp.maximum(zi - mid, 0.0)

        fv = plsc.parallel_loop(
            0, N, step=L, unroll=2, carry=jnp.zeros((L,), jnp.float32)
        )(fb)
        pos = jnp.sum(fv) - 1.0 >= 0.0
        return (jnp.where(pos, mid, lo), jnp.where(pos, hi, mid))

    lo, _hi = lax.fori_loop(0, _N_BISECT, _bisect, (lo2, lo2 + 1.0))

    def _newton(_i, tau):
        S, k = _support(tau)
        return (S - 1.0) * _recip(jnp.maximum(k, 1.0))

    tau = lax.fori_loop(0, _N_NEWTON, _newton, lo)

    q0 = q * NQ

    @pl.loop(0, NQ, step=L)
    def _(i):
        t1[pl.ds(i, L)] = float(N) * jnp.maximum(y[pl.ds(q0 + i, L)] - tau, 0.0)

    pltpu.sync_copy(t1, out_hbm.at[pl.ds(g * N + q0, NQ)])


@jax.jit
def kernel(x, edge_index):
    se = edge_index.astype(jnp.int32).transpose(0, 2, 1)
    mesh = plsc.VectorSubcoreMesh(core_axis_name="c", subcore_axis_name="s")
    cp = pltpu.CompilerParams()
    if "needs_layout_passes" in pltpu.CompilerParams.__dataclass_fields__:
        cp = dataclasses.replace(cp, needs_layout_passes=False)
    run = pl.kernel(
        _sc_body,
        out_type=jax.ShapeDtypeStruct((B * N,), jnp.float32),
        mesh=mesh,
        scratch_types=[
            pltpu.VMEM((N,), jnp.float32),
            pltpu.VMEM((N,), jnp.float32),
            pltpu.VMEM((N,), jnp.float32),
            pltpu.VMEM((NQ,), jnp.float32),
            pltpu.VMEM((4 * N,), jnp.float32),
            pltpu.VMEM((EQ,), jnp.int32),
            pltpu.VMEM((EQ,), jnp.int32),
            pltpu.VMEM_SHARED((2, 4, 4 * N), jnp.float32),
            pltpu.SemaphoreType.DMA,
        ],
        compiler_params=cp,
    )
    return run(x, se)
